# baseline (speedup 1.0000x reference)
import functools

import jax
import jax.numpy as jnp
from jax import lax
from jax.experimental import pallas as pl
from jax.experimental.pallas import tpu as pltpu
from jax.experimental.pallas import tpu_sc as plsc

EMBED_DIM = 64

_info = plsc.get_sparse_core_info()
_NC, _NS = _info.num_cores, _info.num_subcores
_NW = _NC * _NS

_CHUNKS = ((0, 104), (104, 96))
_IMG_G = 4
_NBUF = 2


def _embed_kernel(n_img: int, seq: int):
  img_per_w = n_img // _NW
  n_groups = img_per_w // _IMG_G
  mesh = plsc.VectorSubcoreMesh(core_axis_name="c", subcore_axis_name="s")

  @functools.partial(
      pl.kernel,
      out_type=jax.ShapeDtypeStruct((n_img, seq, EMBED_DIM), jnp.float32),
      mesh=mesh,
      scratch_types=[
          pltpu.VMEM((img_per_w, seq), jnp.int32),
          pltpu.VMEM((_IMG_G, seq, EMBED_DIM), jnp.float32),
          pltpu.VMEM((_IMG_G, seq, EMBED_DIM), jnp.float32),
          pltpu.SemaphoreType.DMA,
          pltpu.SemaphoreType.DMA,
          pltpu.SemaphoreType.DMA,
          pltpu.SemaphoreType.DMA,
      ],
      compiler_params=pltpu.CompilerParams(use_tc_tiling_on_sc=False),
  )
  def body(x_hbm, table_hbm, out_hbm, idx_v, rows_a, rows_b,
           sem_ga, sem_gb, sem_sa, sem_sb):
    wid = lax.axis_index("s") * _NC + lax.axis_index("c")
    img0 = wid * img_per_w
    rows = (rows_a, rows_b)
    sem_g = (sem_ga, sem_gb)
    sem_s = (sem_sa, sem_sb)

    pltpu.sync_copy(x_hbm.at[pl.ds(img0, img_per_w)], idx_v)

    def fire_gathers(g, b):
      hs = []
      for i in range(_IMG_G):
        img = g * _IMG_G + i
        for off, ln in _CHUNKS:
          hs.append(pltpu.async_copy(
              table_hbm.at[idx_v.at[img, pl.ds(off, ln)]],
              rows[b].at[i, pl.ds(off, ln)],
              sem_g[b]))
      return hs

    def drain(hs):
      for h in hs:
        h.wait()

    def fire_store(g, b):
      pltpu.async_copy(rows[b], out_hbm.at[pl.ds(img0 + g * _IMG_G, _IMG_G)],
                       sem_s[b])

    def wait_store(b):
      pltpu.make_async_copy(rows[b], out_hbm.at[pl.ds(img0, _IMG_G)],
                            sem_s[b]).wait()

    for b in range(_NBUF):
      drain(fire_gathers(b, b))
      fire_store(b, b)

    def step(i, carry):
      for b in range(_NBUF):
        g = i * _NBUF + b
        wait_store(b)
        drain(fire_gathers(g, b))
        fire_store(g, b)
      return carry

    lax.fori_loop(1, n_groups // _NBUF, step, 0)

    for b in range(_NBUF):
      wait_store(b)

  return body


def kernel(x, table):
  n_img, seq = x.shape
  return _embed_kernel(n_img, seq)(x.astype(jnp.int32), table)


# speedup vs baseline: 1.3082x; 1.3082x over previous
import functools

import jax
import jax.numpy as jnp
from jax import lax
from jax.experimental import pallas as pl
from jax.experimental.pallas import tpu as pltpu
from jax.experimental.pallas import tpu_sc as plsc

EMBED_DIM = 64
LANES = 128

_info = plsc.get_sparse_core_info()
_NC, _NS = _info.num_cores, _info.num_subcores
_NW = _NC * _NS

_CHUNK = 128
_K = 2
_GROUP = _K * _CHUNK
_NBUF = 2


def _embed_kernel(n_rows: int):
  b_per_w = n_rows // _NW
  n_chunks = b_per_w // _CHUNK
  n_groups = b_per_w // _GROUP
  mesh = plsc.VectorSubcoreMesh(core_axis_name="c", subcore_axis_name="s")

  @functools.partial(
      pl.kernel,
      out_type=jax.ShapeDtypeStruct((n_rows, LANES), jnp.float32),
      mesh=mesh,
      scratch_types=[
          pltpu.VMEM((n_chunks, _CHUNK), jnp.int32),
          pltpu.VMEM((_GROUP, LANES), jnp.float32),
          pltpu.VMEM((_GROUP, LANES), jnp.float32),
          pltpu.SemaphoreType.DMA,
          pltpu.SemaphoreType.DMA,
          pltpu.SemaphoreType.DMA,
          pltpu.SemaphoreType.DMA,
      ],
      compiler_params=pltpu.CompilerParams(use_tc_tiling_on_sc=True),
  )
  def body(idx_hbm, table_hbm, out_hbm, idx_v, rows_a, rows_b,
           sem_ga, sem_gb, sem_sa, sem_sb):
    wid = lax.axis_index("s") * _NC + lax.axis_index("c")
    base = wid * b_per_w
    rows = (rows_a, rows_b)
    sem_g = (sem_ga, sem_gb)
    sem_s = (sem_sa, sem_sb)

    pltpu.sync_copy(idx_hbm.at[pl.ds(wid * n_chunks, n_chunks)], idx_v)

    def fire_gathers(g, b):
      hs = []
      for j in range(_K):
        c = g * _K + j
        hs.append(pltpu.async_copy(
            table_hbm.at[idx_v.at[c]],
            rows[b].at[pl.ds(j * _CHUNK, _CHUNK)],
            sem_g[b]))
      return hs

    def drain(hs):
      for h in hs:
        h.wait()

    def fire_store(g, b):
      pltpu.async_copy(rows[b], out_hbm.at[pl.ds(base + g * _GROUP, _GROUP)],
                       sem_s[b])

    def wait_store(b):
      pltpu.make_async_copy(rows[b], out_hbm.at[pl.ds(base, _GROUP)],
                            sem_s[b]).wait()

    for b in range(_NBUF):
      drain(fire_gathers(b, b))
      fire_store(b, b)

    def step(i, carry):
      for b in range(_NBUF):
        g = i * _NBUF + b
        wait_store(b)
        drain(fire_gathers(g, b))
        fire_store(g, b)
      return carry

    lax.fori_loop(1, n_groups // _NBUF, step, 0)

    for b in range(_NBUF):
      wait_store(b)

  return body


def kernel(x, table):
  n_img, seq = x.shape
  n_rows = n_img * seq
  flat = x.reshape(n_rows // _CHUNK, _CHUNK).astype(jnp.int32)
  table_p = jnp.pad(table, ((0, 0), (0, LANES - EMBED_DIM)))
  out = _embed_kernel(n_rows)(flat, table_p)
  return out.reshape(n_img, seq, LANES)[:, :, :EMBED_DIM]
